# baseline (speedup 1.0000x reference)
import functools

import jax
import jax.numpy as jnp
from jax import lax
from jax.experimental import pallas as pl
from jax.experimental.pallas import tpu as pltpu
from jax.experimental.pallas import tpu_sc as plsc

N = 10000
E = 320000
NODE_DIM = 3
EDGE_DIM = 32
HID = 128
NUM_GRAPHS = 64

NC = 2
NS = 16
NW = NC * NS
EW = E // NW
BLK = 40
NB = EW // BLK

_LOG1P_C = (
    6.944574454165033e-05, 0.9962619482337974, -0.46644243862758844,
    0.2186654836622489, -0.05545931374208889,
)


def _log1p01(a):
    acc = jnp.full_like(a, _LOG1P_C[-1])
    for c in _LOG1P_C[-2::-1]:
        acc = acc * a + jnp.float32(c)
    return acc


def _gate(f, s):
    t = 1.0 + jnp.exp(-f)
    sp = jnp.maximum(s, 0.0) + _log1p01(jnp.exp(-jnp.abs(s)))
    return sp / t


@functools.lru_cache(maxsize=None)
def _make_edge_sc(conv1, name):
    CH = 1 if conv1 else 8
    if conv1:
        gshape = [(2, BLK, 128), (2, BLK, 128), (2, BLK, 32)]
        gdt = jnp.float32
    else:
        gshape = [(2, BLK, 128), (2, BLK, 128), (2, BLK, 128)]
        gdt = jnp.int32
    mesh = plsc.VectorSubcoreMesh(core_axis_name="c", subcore_axis_name="s",
                                  num_cores=NC, num_subcores=NS)

    @functools.partial(
        pl.kernel,
        out_type=jax.ShapeDtypeStruct((NC, N, HID), jnp.float32),
        mesh=mesh,
        scratch_types=[
            pltpu.VMEM((3, BLK), jnp.int32),
            pltpu.VMEM((3, BLK), jnp.int32),
            pltpu.VMEM(gshape[0], gdt),
            pltpu.VMEM(gshape[1], gdt),
            pltpu.VMEM(gshape[2], gdt),
            pltpu.VMEM((BLK, HID), jnp.float32),
            pltpu.VMEM_SHARED((N, HID), jnp.float32),
            pltpu.SemaphoreType.DMA,
            pltpu.SemaphoreType.DMA,
            pltpu.SemaphoreType.DMA,
            pltpu.SemaphoreType.DMA,
            pltpu.SemaphoreType.DMA,
        ],
        name=name,
        compiler_params=pltpu.CompilerParams(needs_layout_passes=False),
    )
    def k(td_hbm, ts_hbm, eq_hbm, dst_hbm, src_hbm, zero_hbm, out_hbm,
          dvi, svi, rdv, rsv, eqv, msgv, aggsh, sid_, sis_, sd, ss, se):
        cid = lax.axis_index("c")
        sid = lax.axis_index("s")
        wid = cid * NS + sid
        base = wid * EW

        @pl.when(sid == 0)
        def _zero():
            pltpu.sync_copy(zero_hbm, aggsh)

        if conv1:
            def zrow(r, rc):
                for c in range(1, 8):
                    msgv[r, pl.ds(c * 16, 16)] = jnp.zeros((16,), jnp.float32)
                return rc
            lax.fori_loop(0, BLK, zrow, 0)

        plsc.subcore_barrier()

        def block(k_, carry):
            j1 = k_ - 1
            j2 = k_ - 2

            @pl.when(jnp.logical_and(j1 >= 0, j1 < NB))
            def _w_idx():
                s3 = lax.rem(j1, 3)
                off = pl.multiple_of(base + j1 * BLK, 8)
                pltpu.make_async_copy(dst_hbm.at[pl.ds(off, BLK)],
                                      dvi.at[s3], sid_).wait()
                pltpu.make_async_copy(src_hbm.at[pl.ds(off, BLK)],
                                      svi.at[s3], sis_).wait()

            @pl.when(k_ < NB)
            def _i_idx():
                s3 = lax.rem(k_, 3)
                off = pl.multiple_of(base + k_ * BLK, 8)
                pltpu.async_copy(dst_hbm.at[pl.ds(off, BLK)], dvi.at[s3],
                                 sid_)
                pltpu.async_copy(src_hbm.at[pl.ds(off, BLK)], svi.at[s3],
                                 sis_)

            @pl.when(j2 >= 0)
            def _w_gat():
                s3 = lax.rem(j2, 3)
                s2 = lax.rem(j2, 2)
                off = pl.multiple_of(base + j2 * BLK, 8)
                pltpu.make_async_copy(td_hbm.at[dvi.at[s3]], rdv.at[s2],
                                      sd).wait()
                pltpu.make_async_copy(ts_hbm.at[svi.at[s3]], rsv.at[s2],
                                      ss).wait()
                pltpu.make_async_copy(eq_hbm.at[pl.ds(off, BLK)], eqv.at[s2],
                                      se).wait()

            @pl.when(jnp.logical_and(j1 >= 0, j1 < NB))
            def _i_gat():
                s3 = lax.rem(j1, 3)
                s2 = lax.rem(j1, 2)
                off = pl.multiple_of(base + j1 * BLK, 8)
                pltpu.async_copy(td_hbm.at[dvi.at[s3]], rdv.at[s2], sd)
                pltpu.async_copy(ts_hbm.at[svi.at[s3]], rsv.at[s2], ss)
                pltpu.async_copy(eq_hbm.at[pl.ds(off, BLK)], eqv.at[s2], se)

            @pl.when(j2 >= 0)
            def _c():
                s3 = lax.rem(j2, 3)
                s2 = lax.rem(j2, 2)

                def row(r, rc):
                    for c in range(CH):
                        lo = c * 16
                        if conv1:
                            f = (rdv[s2, r, pl.ds(0, 16)]
                                 + rsv[s2, r, pl.ds(0, 16)]
                                 + eqv[s2, r, pl.ds(0, 16)])
                            s = (rdv[s2, r, pl.ds(64, 16)]
                                 + rsv[s2, r, pl.ds(64, 16)]
                                 + eqv[s2, r, pl.ds(16, 16)])
                        else:
                            wd_ = rdv[s2, r, pl.ds(lo, 16)]
                            ws_ = rsv[s2, r, pl.ds(lo, 16)]
                            we_ = eqv[s2, r, pl.ds(lo, 16)]
                            hi = jnp.int32(-65536)

                            def bflo(w):
                                return plsc.bitcast(
                                    lax.shift_left(w, 16), jnp.float32)

                            def bfhi(w):
                                return plsc.bitcast(
                                    jnp.bitwise_and(w, hi), jnp.float32)

                            f = bflo(wd_) + bflo(ws_) + bflo(we_)
                            s = bfhi(wd_) + bfhi(ws_) + bfhi(we_)
                        msgv[r, pl.ds(lo, 16)] = _gate(f, s)
                    return rc

                lax.fori_loop(0, BLK, row, 0)
                pltpu.sync_copy(msgv, aggsh.at[dvi.at[s3]], add=True)

            return carry

        lax.fori_loop(0, NB + 2, block, 0)
        plsc.subcore_barrier()

        @pl.when(sid == 0)
        def _flush():
            pltpu.sync_copy(aggsh, out_hbm.at[cid])

    return k


def _mm_body(x_ref, w_ref, b_ref, o_ref):
    acc = jnp.dot(x_ref[...], w_ref[...], preferred_element_type=jnp.float32)
    o_ref[...] = (acc + b_ref[...]).astype(o_ref.dtype)


def _mm(x, w, b, blk, out_dt=jnp.float32):
    m, k = x.shape
    n = w.shape[1]
    return pl.pallas_call(
        _mm_body,
        grid=(m // blk,),
        in_specs=[
            pl.BlockSpec((blk, k), lambda i: (i, 0)),
            pl.BlockSpec((k, n), lambda i: (0, 0)),
            pl.BlockSpec((1, n), lambda i: (0, 0)),
        ],
        out_specs=pl.BlockSpec((blk, n), lambda i: (i, 0)),
        out_shape=jax.ShapeDtypeStruct((m, n), out_dt),
    )(x, w, b.reshape(1, n))


def _finish_body(ncut, x_ref, aa_ref, ab_ref, wp_ref, bp_ref, wd_ref, ws_ref,
                 h_ref, td_ref, ts_ref):
    h1 = x_ref[...] + aa_ref[:, :ncut] + ab_ref[:, :ncut]
    h = jnp.maximum(
        jnp.dot(h1, wp_ref[...], preferred_element_type=jnp.float32)
        + bp_ref[...], 0.0)
    h_ref[...] = h
    td_ref[...] = jnp.dot(
        h, wd_ref[...], preferred_element_type=jnp.float32).astype(
            td_ref.dtype)
    ts_ref[...] = jnp.dot(
        h, ws_ref[...], preferred_element_type=jnp.float32).astype(
            ts_ref.dtype)


def _finish(x, aggA, aggB, wp, bp, wd, ws, blk):
    m, d = x.shape
    wa = aggA.shape[1]
    kp, n = wp.shape
    n2 = wd.shape[1]
    return pl.pallas_call(
        functools.partial(_finish_body, d),
        grid=(m // blk,),
        in_specs=[
            pl.BlockSpec((blk, d), lambda i: (i, 0)),
            pl.BlockSpec((blk, wa), lambda i: (i, 0)),
            pl.BlockSpec((blk, wa), lambda i: (i, 0)),
            pl.BlockSpec((kp, n), lambda i: (0, 0)),
            pl.BlockSpec((1, n), lambda i: (0, 0)),
            pl.BlockSpec((n, n2), lambda i: (0, 0)),
            pl.BlockSpec((n, n2), lambda i: (0, 0)),
        ],
        out_specs=[
            pl.BlockSpec((blk, n), lambda i: (i, 0)),
            pl.BlockSpec((blk, n2), lambda i: (i, 0)),
            pl.BlockSpec((blk, n2), lambda i: (i, 0)),
        ],
        out_shape=[
            jax.ShapeDtypeStruct((m, n), jnp.float32),
            jax.ShapeDtypeStruct((m, n2), jnp.bfloat16),
            jax.ShapeDtypeStruct((m, n2), jnp.bfloat16),
        ],
    )(x, aggA, aggB, wp, bp.reshape(1, n), wd, ws)


def _final_body(h_ref, aa_ref, ab_ref, bt_ref, wfc_ref, bfc_ref, wh_ref,
                bh_ref, o_ref, sacc, cacc):
    i = pl.program_id(0)
    nblk = pl.num_programs(0)

    @pl.when(i == 0)
    def _init():
        sacc[...] = jnp.zeros_like(sacc)
        cacc[...] = jnp.zeros_like(cacc)

    h3 = jnp.maximum(h_ref[...] + aa_ref[...] + ab_ref[...], 0.0)
    ids = lax.broadcasted_iota(jnp.int32, (h3.shape[0], NUM_GRAPHS), 1)
    oh = (bt_ref[...] == ids).astype(jnp.float32)
    sacc[...] += lax.dot_general(oh, h3, (((0,), (0,)), ((), ())),
                                 preferred_element_type=jnp.float32)
    cnt = jnp.sum(oh, axis=0)
    cacc[...] += jnp.broadcast_to(cnt[:, None], cacc.shape)

    @pl.when(i == nblk - 1)
    def _emit():
        pooled = sacc[...] / jnp.maximum(cacc[...], 1.0)
        g = jnp.maximum(
            jnp.dot(pooled, wfc_ref[...], preferred_element_type=jnp.float32)
            + bfc_ref[...], 0.0)
        o_ref[...] = (jnp.dot(g, wh_ref[...], preferred_element_type=jnp.float32)
                      + bh_ref[...])


def _final(h, aggA, aggB, batch2d, wfc, bfc, wh, bh, blk):
    m = h.shape[0]
    nh = wh.shape[1]
    return pl.pallas_call(
        _final_body,
        grid=(m // blk,),
        in_specs=[
            pl.BlockSpec((blk, HID), lambda i: (i, 0)),
            pl.BlockSpec((blk, HID), lambda i: (i, 0)),
            pl.BlockSpec((blk, HID), lambda i: (i, 0)),
            pl.BlockSpec((blk, 1), lambda i: (i, 0)),
            pl.BlockSpec((HID, HID), lambda i: (0, 0)),
            pl.BlockSpec((1, HID), lambda i: (0, 0)),
            pl.BlockSpec((HID, nh), lambda i: (0, 0)),
            pl.BlockSpec((1, nh), lambda i: (0, 0)),
        ],
        out_specs=pl.BlockSpec((NUM_GRAPHS, nh), lambda i: (0, 0)),
        out_shape=jax.ShapeDtypeStruct((NUM_GRAPHS, nh), jnp.float32),
        scratch_shapes=[
            pltpu.VMEM((NUM_GRAPHS, HID), jnp.float32),
            pltpu.VMEM((NUM_GRAPHS, HID), jnp.float32),
        ],
    )(h, aggA, aggB, batch2d, wfc, bfc.reshape(1, HID), wh, bh.reshape(1, nh))


def kernel(x, edge_index, edge_attr, batch,
           Wf1, bf1, Ws1, bs1, Wp, bp,
           Wf2, bf2, Ws2, bs2, Wf3, bf3, Ws3, bs3,
           Wfc, bfc, Wh, bh):
    f32 = jnp.float32
    dst = edge_index[1]
    src = edge_index[0]

    w1d = jnp.zeros((NODE_DIM, HID), f32)
    w1d = w1d.at[:, 0:3].set(Wf1[0:3]).at[:, 64:67].set(Ws1[0:3])
    w1s = jnp.zeros((NODE_DIM, HID), f32)
    w1s = w1s.at[:, 0:3].set(Wf1[3:6]).at[:, 64:67].set(Ws1[3:6])
    w1e = jnp.zeros((EDGE_DIM, 32), f32)
    w1e = w1e.at[:, 0:3].set(Wf1[6:38]).at[:, 16:19].set(Ws1[6:38])
    b1q = jnp.zeros((32,), f32).at[0:3].set(bf1).at[16:19].set(bs1)

    def ilv(a, b):
        return jnp.stack([a, b], axis=-1).reshape(a.shape[0], -1)

    wd2 = ilv(Wf2[0:HID], Ws2[0:HID])
    ws2 = ilv(Wf2[HID:2 * HID], Ws2[HID:2 * HID])
    we2 = ilv(Wf2[2 * HID:], Ws2[2 * HID:])
    bq2 = jnp.stack([bf2, bs2], axis=-1).reshape(-1)
    wd3 = ilv(Wf3[0:HID], Ws3[0:HID])
    ws3 = ilv(Wf3[HID:2 * HID], Ws3[HID:2 * HID])
    we3 = ilv(Wf3[2 * HID:], Ws3[2 * HID:])
    bq3 = jnp.stack([bf3, bs3], axis=-1).reshape(-1)

    z128 = jnp.zeros((N, HID), f32)

    t1d = _mm(x, w1d, jnp.zeros((HID,), f32), 1000)
    t1s = _mm(x, w1s, jnp.zeros((HID,), f32), 1000)
    eq1 = _mm(edge_attr, w1e, b1q, 4000)
    agg1 = _make_edge_sc(True, "edge_sc_conv1")(t1d, t1s, eq1, dst, src, z128)

    def p32(a):
        return lax.bitcast_convert_type(
            a.reshape(a.shape[0], -1, 2), jnp.int32)

    h, td2, ts2 = _finish(x, agg1[0], agg1[1], Wp, bp, wd2, ws2, 2000)
    eq2 = _mm(edge_attr, we2, bq2, 4000, jnp.bfloat16)
    agg2 = _make_edge_sc(False, "edge_sc_hid")(
        p32(td2), p32(ts2), p32(eq2), dst, src, z128)

    h2, td3, ts3 = _finish_hid(h, agg2[0], agg2[1], wd3, ws3, 2000)
    eq3 = _mm(edge_attr, we3, bq3, 4000, jnp.bfloat16)
    agg3 = _make_edge_sc(False, "edge_sc_hid")(
        p32(td3), p32(ts3), p32(eq3), dst, src, z128)

    return _final(h2, agg3[0], agg3[1], batch.reshape(N, 1),
                  Wfc, bfc, Wh, bh, 1000)


def _finish_hid_body(h_ref, aa_ref, ab_ref, wd_ref, ws_ref,
                     h2_ref, td_ref, ts_ref):
    h2 = jnp.maximum(h_ref[...] + aa_ref[...] + ab_ref[...], 0.0)
    h2_ref[...] = h2
    td_ref[...] = jnp.dot(
        h2, wd_ref[...], preferred_element_type=jnp.float32).astype(
            td_ref.dtype)
    ts_ref[...] = jnp.dot(
        h2, ws_ref[...], preferred_element_type=jnp.float32).astype(
            ts_ref.dtype)


def _finish_hid(h, aggA, aggB, wd, ws, blk):
    m = h.shape[0]
    n2 = wd.shape[1]
    return pl.pallas_call(
        _finish_hid_body,
        grid=(m // blk,),
        in_specs=[
            pl.BlockSpec((blk, HID), lambda i: (i, 0)),
            pl.BlockSpec((blk, HID), lambda i: (i, 0)),
            pl.BlockSpec((blk, HID), lambda i: (i, 0)),
            pl.BlockSpec((HID, n2), lambda i: (0, 0)),
            pl.BlockSpec((HID, n2), lambda i: (0, 0)),
        ],
        out_specs=[
            pl.BlockSpec((blk, HID), lambda i: (i, 0)),
            pl.BlockSpec((blk, n2), lambda i: (i, 0)),
            pl.BlockSpec((blk, n2), lambda i: (i, 0)),
        ],
        out_shape=[
            jax.ShapeDtypeStruct((m, HID), jnp.float32),
            jax.ShapeDtypeStruct((m, n2), jnp.bfloat16),
            jax.ShapeDtypeStruct((m, n2), jnp.bfloat16),
        ],
    )(h, aggA, aggB, wd, ws)


# speedup vs baseline: 1.6569x; 1.6569x over previous
import functools

import jax
import jax.numpy as jnp
from jax import lax
from jax.experimental import pallas as pl
from jax.experimental.pallas import tpu as pltpu
from jax.experimental.pallas import tpu_sc as plsc

N = 10000
E = 320000
NODE_DIM = 3
EDGE_DIM = 32
HID = 128
NUM_GRAPHS = 64

NC = 2
NS = 16
NW = NC * NS
EW = E // NW
BLK = 40
NB = EW // BLK

_LOG1P_C = (
    6.944574454165033e-05, 0.9962619482337974, -0.46644243862758844,
    0.2186654836622489, -0.05545931374208889,
)


def _log1p01(a):
    acc = jnp.full_like(a, _LOG1P_C[-1])
    for c in _LOG1P_C[-2::-1]:
        acc = acc * a + jnp.float32(c)
    return acc


def _gate(f, s):
    t = 1.0 + jnp.exp(-f)
    sp = jnp.maximum(s, 0.0) + _log1p01(jnp.exp(-jnp.abs(s)))
    return sp / t


@functools.lru_cache(maxsize=None)
def _make_edge_sc(conv1, name):
    CH = 1 if conv1 else 8
    if conv1:
        gshape = [(2, BLK, 128), (2, BLK, 128), (2, BLK, 32)]
        gdt = jnp.float32
    else:
        gshape = [(2, BLK, 128), (2, BLK, 128), (2, BLK, 128)]
        gdt = jnp.int32
    mesh = plsc.VectorSubcoreMesh(core_axis_name="c", subcore_axis_name="s",
                                  num_cores=NC, num_subcores=NS)

    @functools.partial(
        pl.kernel,
        out_type=jax.ShapeDtypeStruct((NC, N, HID), jnp.float32),
        mesh=mesh,
        scratch_types=[
            pltpu.VMEM((3, BLK), jnp.int32),
            pltpu.VMEM((3, BLK), jnp.int32),
            pltpu.VMEM(gshape[0], gdt),
            pltpu.VMEM(gshape[1], gdt),
            pltpu.VMEM(gshape[2], gdt),
            pltpu.VMEM((BLK, HID), jnp.float32),
            pltpu.VMEM_SHARED((N, HID), jnp.float32),
            pltpu.SemaphoreType.DMA,
            pltpu.SemaphoreType.DMA,
            pltpu.SemaphoreType.DMA,
            pltpu.SemaphoreType.DMA,
            pltpu.SemaphoreType.DMA,
        ],
        name=name,
        compiler_params=pltpu.CompilerParams(needs_layout_passes=False),
    )
    def k(td_hbm, ts_hbm, eq_hbm, dst_hbm, src_hbm, zero_hbm, out_hbm,
          dvi, svi, rdv, rsv, eqv, msgv, aggsh, sid_, sis_, sd, ss, se):
        cid = lax.axis_index("c")
        sid = lax.axis_index("s")
        wid = cid * NS + sid
        base = wid * EW

        @pl.when(sid == 0)
        def _zero():
            pltpu.sync_copy(zero_hbm, aggsh)

        if conv1:
            def zrow(r, rc):
                for c in range(1, 8):
                    msgv[r, pl.ds(c * 16, 16)] = jnp.zeros((16,), jnp.float32)
                return rc
            lax.fori_loop(0, BLK, zrow, 0)

        plsc.subcore_barrier()

        def block(k_, carry):
            j1 = k_ - 1
            j2 = k_ - 2

            @pl.when(jnp.logical_and(j1 >= 0, j1 < NB))
            def _w_idx():
                s3 = lax.rem(j1, 3)
                off = pl.multiple_of(base + j1 * BLK, 8)
                pltpu.make_async_copy(dst_hbm.at[pl.ds(off, BLK)],
                                      dvi.at[s3], sid_).wait()
                pltpu.make_async_copy(src_hbm.at[pl.ds(off, BLK)],
                                      svi.at[s3], sis_).wait()

            @pl.when(k_ < NB)
            def _i_idx():
                s3 = lax.rem(k_, 3)
                off = pl.multiple_of(base + k_ * BLK, 8)
                pltpu.async_copy(dst_hbm.at[pl.ds(off, BLK)], dvi.at[s3],
                                 sid_)
                pltpu.async_copy(src_hbm.at[pl.ds(off, BLK)], svi.at[s3],
                                 sis_)

            @pl.when(j2 >= 0)
            def _w_gat():
                s3 = lax.rem(j2, 3)
                s2 = lax.rem(j2, 2)
                off = pl.multiple_of(base + j2 * BLK, 8)
                pltpu.make_async_copy(td_hbm.at[dvi.at[s3]], rdv.at[s2],
                                      sd).wait()
                pltpu.make_async_copy(ts_hbm.at[svi.at[s3]], rsv.at[s2],
                                      ss).wait()
                pltpu.make_async_copy(eq_hbm.at[pl.ds(off, BLK)], eqv.at[s2],
                                      se).wait()

            @pl.when(jnp.logical_and(j1 >= 0, j1 < NB))
            def _i_gat():
                s3 = lax.rem(j1, 3)
                s2 = lax.rem(j1, 2)
                off = pl.multiple_of(base + j1 * BLK, 8)
                pltpu.async_copy(td_hbm.at[dvi.at[s3]], rdv.at[s2], sd)
                pltpu.async_copy(ts_hbm.at[svi.at[s3]], rsv.at[s2], ss)
                pltpu.async_copy(eq_hbm.at[pl.ds(off, BLK)], eqv.at[s2], se)

            @pl.when(j2 >= 0)
            def _c():
                s3 = lax.rem(j2, 3)
                s2 = lax.rem(j2, 2)

                def row(r, rc):
                    for c in range(CH):
                        lo = c * 16
                        if conv1:
                            f = (rdv[s2, r, pl.ds(0, 16)]
                                 + rsv[s2, r, pl.ds(0, 16)]
                                 + eqv[s2, r, pl.ds(0, 16)])
                            s = (rdv[s2, r, pl.ds(64, 16)]
                                 + rsv[s2, r, pl.ds(64, 16)]
                                 + eqv[s2, r, pl.ds(16, 16)])
                        else:
                            wd_ = rdv[s2, r, pl.ds(lo, 16)]
                            ws_ = rsv[s2, r, pl.ds(lo, 16)]
                            we_ = eqv[s2, r, pl.ds(lo, 16)]
                            hi = jnp.int32(-65536)

                            def bflo(w):
                                return plsc.bitcast(
                                    lax.shift_left(w, 16), jnp.float32)

                            def bfhi(w):
                                return plsc.bitcast(
                                    jnp.bitwise_and(w, hi), jnp.float32)

                            f = bflo(wd_) + bflo(ws_) + bflo(we_)
                            s = bfhi(wd_) + bfhi(ws_) + bfhi(we_)
                        msgv[r, pl.ds(lo, 16)] = _gate(f, s)
                    return rc

                lax.fori_loop(0, BLK, row, 0)
                pltpu.sync_copy(msgv, aggsh.at[dvi.at[s3]], add=True)

            return carry

        lax.fori_loop(0, NB + 2, block, 0)
        plsc.subcore_barrier()

        @pl.when(sid == 0)
        def _flush():
            pltpu.sync_copy(aggsh, out_hbm.at[cid])

    return k


def _pack16(f, s):
    bfi = lax.bitcast_convert_type(
        f.astype(jnp.bfloat16).astype(jnp.float32), jnp.int32)
    bsi = lax.bitcast_convert_type(
        s.astype(jnp.bfloat16).astype(jnp.float32), jnp.int32)
    return jnp.bitwise_or(jnp.bitwise_and(bsi, jnp.int32(-65536)),
                          lax.shift_right_logical(bfi, 16))


def _mm_body(x_ref, w_ref, b_ref, o_ref):
    acc = jnp.dot(x_ref[...], w_ref[...], preferred_element_type=jnp.float32)
    acc = acc + b_ref[...]
    if o_ref.dtype == jnp.int32:
        n2 = acc.shape[1] // 2
        o_ref[...] = _pack16(acc[:, :n2], acc[:, n2:])
    else:
        o_ref[...] = acc.astype(o_ref.dtype)


def _mm(x, w, b, blk, out_dt=jnp.float32):
    m, k = x.shape
    n = w.shape[1]
    no = n // 2 if out_dt == jnp.int32 else n
    return pl.pallas_call(
        _mm_body,
        grid=(m // blk,),
        in_specs=[
            pl.BlockSpec((blk, k), lambda i: (i, 0)),
            pl.BlockSpec((k, n), lambda i: (0, 0)),
            pl.BlockSpec((1, n), lambda i: (0, 0)),
        ],
        out_specs=pl.BlockSpec((blk, no), lambda i: (i, 0)),
        out_shape=jax.ShapeDtypeStruct((m, no), out_dt),
    )(x, w, b.reshape(1, n))


def _finish_body(ncut, x_ref, aa_ref, ab_ref, wp_ref, bp_ref, wd_ref, ws_ref,
                 h_ref, td_ref, ts_ref):
    h1 = x_ref[...] + aa_ref[:, :ncut] + ab_ref[:, :ncut]
    h = jnp.maximum(
        jnp.dot(h1, wp_ref[...], preferred_element_type=jnp.float32)
        + bp_ref[...], 0.0)
    h_ref[...] = h
    td = jnp.dot(h, wd_ref[...], preferred_element_type=jnp.float32)
    ts = jnp.dot(h, ws_ref[...], preferred_element_type=jnp.float32)
    td_ref[...] = _pack16(td[:, :HID], td[:, HID:])
    ts_ref[...] = _pack16(ts[:, :HID], ts[:, HID:])


def _finish(x, aggA, aggB, wp, bp, wd, ws, blk):
    m, d = x.shape
    wa = aggA.shape[1]
    kp, n = wp.shape
    n2 = wd.shape[1]
    return pl.pallas_call(
        functools.partial(_finish_body, d),
        grid=(m // blk,),
        in_specs=[
            pl.BlockSpec((blk, d), lambda i: (i, 0)),
            pl.BlockSpec((blk, wa), lambda i: (i, 0)),
            pl.BlockSpec((blk, wa), lambda i: (i, 0)),
            pl.BlockSpec((kp, n), lambda i: (0, 0)),
            pl.BlockSpec((1, n), lambda i: (0, 0)),
            pl.BlockSpec((n, n2), lambda i: (0, 0)),
            pl.BlockSpec((n, n2), lambda i: (0, 0)),
        ],
        out_specs=[
            pl.BlockSpec((blk, n), lambda i: (i, 0)),
            pl.BlockSpec((blk, n2 // 2), lambda i: (i, 0)),
            pl.BlockSpec((blk, n2 // 2), lambda i: (i, 0)),
        ],
        out_shape=[
            jax.ShapeDtypeStruct((m, n), jnp.float32),
            jax.ShapeDtypeStruct((m, n2 // 2), jnp.int32),
            jax.ShapeDtypeStruct((m, n2 // 2), jnp.int32),
        ],
    )(x, aggA, aggB, wp, bp.reshape(1, n), wd, ws)


def _final_body(h_ref, aa_ref, ab_ref, bt_ref, wfc_ref, bfc_ref, wh_ref,
                bh_ref, o_ref, sacc, cacc):
    i = pl.program_id(0)
    nblk = pl.num_programs(0)

    @pl.when(i == 0)
    def _init():
        sacc[...] = jnp.zeros_like(sacc)
        cacc[...] = jnp.zeros_like(cacc)

    h3 = jnp.maximum(h_ref[...] + aa_ref[...] + ab_ref[...], 0.0)
    ids = lax.broadcasted_iota(jnp.int32, (h3.shape[0], NUM_GRAPHS), 1)
    oh = (bt_ref[...] == ids).astype(jnp.float32)
    sacc[...] += lax.dot_general(oh, h3, (((0,), (0,)), ((), ())),
                                 preferred_element_type=jnp.float32)
    cnt = jnp.sum(oh, axis=0)
    cacc[...] += jnp.broadcast_to(cnt[:, None], cacc.shape)

    @pl.when(i == nblk - 1)
    def _emit():
        pooled = sacc[...] / jnp.maximum(cacc[...], 1.0)
        g = jnp.maximum(
            jnp.dot(pooled, wfc_ref[...], preferred_element_type=jnp.float32)
            + bfc_ref[...], 0.0)
        o_ref[...] = (jnp.dot(g, wh_ref[...], preferred_element_type=jnp.float32)
                      + bh_ref[...])


def _final(h, aggA, aggB, batch2d, wfc, bfc, wh, bh, blk):
    m = h.shape[0]
    nh = wh.shape[1]
    return pl.pallas_call(
        _final_body,
        grid=(m // blk,),
        in_specs=[
            pl.BlockSpec((blk, HID), lambda i: (i, 0)),
            pl.BlockSpec((blk, HID), lambda i: (i, 0)),
            pl.BlockSpec((blk, HID), lambda i: (i, 0)),
            pl.BlockSpec((blk, 1), lambda i: (i, 0)),
            pl.BlockSpec((HID, HID), lambda i: (0, 0)),
            pl.BlockSpec((1, HID), lambda i: (0, 0)),
            pl.BlockSpec((HID, nh), lambda i: (0, 0)),
            pl.BlockSpec((1, nh), lambda i: (0, 0)),
        ],
        out_specs=pl.BlockSpec((NUM_GRAPHS, nh), lambda i: (0, 0)),
        out_shape=jax.ShapeDtypeStruct((NUM_GRAPHS, nh), jnp.float32),
        scratch_shapes=[
            pltpu.VMEM((NUM_GRAPHS, HID), jnp.float32),
            pltpu.VMEM((NUM_GRAPHS, HID), jnp.float32),
        ],
    )(h, aggA, aggB, batch2d, wfc, bfc.reshape(1, HID), wh, bh.reshape(1, nh))


def kernel(x, edge_index, edge_attr, batch,
           Wf1, bf1, Ws1, bs1, Wp, bp,
           Wf2, bf2, Ws2, bs2, Wf3, bf3, Ws3, bs3,
           Wfc, bfc, Wh, bh):
    f32 = jnp.float32
    dst = edge_index[1]
    src = edge_index[0]

    w1d = jnp.zeros((NODE_DIM, HID), f32)
    w1d = w1d.at[:, 0:3].set(Wf1[0:3]).at[:, 64:67].set(Ws1[0:3])
    w1s = jnp.zeros((NODE_DIM, HID), f32)
    w1s = w1s.at[:, 0:3].set(Wf1[3:6]).at[:, 64:67].set(Ws1[3:6])
    w1e = jnp.zeros((EDGE_DIM, 32), f32)
    w1e = w1e.at[:, 0:3].set(Wf1[6:38]).at[:, 16:19].set(Ws1[6:38])
    b1q = jnp.zeros((32,), f32).at[0:3].set(bf1).at[16:19].set(bs1)

    wd2 = jnp.concatenate([Wf2[0:HID], Ws2[0:HID]], axis=1)
    ws2 = jnp.concatenate([Wf2[HID:2 * HID], Ws2[HID:2 * HID]], axis=1)
    we2 = jnp.concatenate([Wf2[2 * HID:], Ws2[2 * HID:]], axis=1)
    bq2 = jnp.concatenate([bf2, bs2])
    wd3 = jnp.concatenate([Wf3[0:HID], Ws3[0:HID]], axis=1)
    ws3 = jnp.concatenate([Wf3[HID:2 * HID], Ws3[HID:2 * HID]], axis=1)
    we3 = jnp.concatenate([Wf3[2 * HID:], Ws3[2 * HID:]], axis=1)
    bq3 = jnp.concatenate([bf3, bs3])

    z128 = jnp.zeros((N, HID), f32)

    t1d = _mm(x, w1d, jnp.zeros((HID,), f32), 1000)
    t1s = _mm(x, w1s, jnp.zeros((HID,), f32), 1000)
    eq1 = _mm(edge_attr, w1e, b1q, 4000)
    agg1 = _make_edge_sc(True, "edge_sc_conv1")(t1d, t1s, eq1, dst, src, z128)

    h, td2, ts2 = _finish(x, agg1[0], agg1[1], Wp, bp, wd2, ws2, 2000)
    eq2 = _mm(edge_attr, we2, bq2, 4000, jnp.int32)
    agg2 = _make_edge_sc(False, "edge_sc_hid")(td2, ts2, eq2, dst, src, z128)

    h2, td3, ts3 = _finish_hid(h, agg2[0], agg2[1], wd3, ws3, 2000)
    eq3 = _mm(edge_attr, we3, bq3, 4000, jnp.int32)
    agg3 = _make_edge_sc(False, "edge_sc_hid")(td3, ts3, eq3, dst, src, z128)

    return _final(h2, agg3[0], agg3[1], batch.reshape(N, 1),
                  Wfc, bfc, Wh, bh, 1000)


def _finish_hid_body(h_ref, aa_ref, ab_ref, wd_ref, ws_ref,
                     h2_ref, td_ref, ts_ref):
    h2 = jnp.maximum(h_ref[...] + aa_ref[...] + ab_ref[...], 0.0)
    h2_ref[...] = h2
    td = jnp.dot(h2, wd_ref[...], preferred_element_type=jnp.float32)
    ts = jnp.dot(h2, ws_ref[...], preferred_element_type=jnp.float32)
    td_ref[...] = _pack16(td[:, :HID], td[:, HID:])
    ts_ref[...] = _pack16(ts[:, :HID], ts[:, HID:])


def _finish_hid(h, aggA, aggB, wd, ws, blk):
    m = h.shape[0]
    n2 = wd.shape[1]
    return pl.pallas_call(
        _finish_hid_body,
        grid=(m // blk,),
        in_specs=[
            pl.BlockSpec((blk, HID), lambda i: (i, 0)),
            pl.BlockSpec((blk, HID), lambda i: (i, 0)),
            pl.BlockSpec((blk, HID), lambda i: (i, 0)),
            pl.BlockSpec((HID, n2), lambda i: (0, 0)),
            pl.BlockSpec((HID, n2), lambda i: (0, 0)),
        ],
        out_specs=[
            pl.BlockSpec((blk, HID), lambda i: (i, 0)),
            pl.BlockSpec((blk, n2 // 2), lambda i: (i, 0)),
            pl.BlockSpec((blk, n2 // 2), lambda i: (i, 0)),
        ],
        out_shape=[
            jax.ShapeDtypeStruct((m, HID), jnp.float32),
            jax.ShapeDtypeStruct((m, n2 // 2), jnp.int32),
            jax.ShapeDtypeStruct((m, n2 // 2), jnp.int32),
        ],
    )(h, aggA, aggB, wd, ws)


# speedup vs baseline: 4.4732x; 2.6997x over previous
import functools

import jax
import jax.numpy as jnp
from jax import lax
from jax.experimental import pallas as pl
from jax.experimental.pallas import tpu as pltpu
from jax.experimental.pallas import tpu_sc as plsc

N = 10000
E = 320000
NODE_DIM = 3
EDGE_DIM = 32
HID = 128
NUM_GRAPHS = 64

NC = 2
NS = 16
NW = NC * NS
EW = E // NW
BLK = 40
NB = EW // BLK

_LOG1P_C = (
    6.944574454165033e-05, 0.9962619482337974, -0.46644243862758844,
    0.2186654836622489, -0.05545931374208889,
)


def _log1p01(a):
    acc = jnp.full_like(a, _LOG1P_C[-1])
    for c in _LOG1P_C[-2::-1]:
        acc = acc * a + jnp.float32(c)
    return acc


def _gate(f, s):
    t = 1.0 + jnp.exp(-f)
    sp = jnp.maximum(s, 0.0) + _log1p01(jnp.exp(-jnp.abs(s)))
    return sp / t


@functools.lru_cache(maxsize=None)
def _make_edge_sc(conv1, name):
    CH = 1 if conv1 else 8
    if conv1:
        gshape = [(2, BLK, 128), (2, BLK, 128), (2, BLK, 32)]
        gdt = jnp.float32
    else:
        gshape = [(2, BLK, 128), (2, BLK, 128), (2, BLK, 128)]
        gdt = jnp.int32
    mesh = plsc.VectorSubcoreMesh(core_axis_name="c", subcore_axis_name="s",
                                  num_cores=NC, num_subcores=NS)

    @functools.partial(
        pl.kernel,
        out_type=jax.ShapeDtypeStruct((NC, N, HID), jnp.float32),
        mesh=mesh,
        scratch_types=[
            pltpu.VMEM((3, BLK), jnp.int32),
            pltpu.VMEM((3, BLK), jnp.int32),
            pltpu.VMEM(gshape[0], gdt),
            pltpu.VMEM(gshape[1], gdt),
            pltpu.VMEM(gshape[2], gdt),
            pltpu.VMEM((BLK, HID), jnp.float32),
            pltpu.VMEM_SHARED((N, HID), jnp.float32),
            pltpu.SemaphoreType.DMA,
            pltpu.SemaphoreType.DMA,
            pltpu.SemaphoreType.DMA,
            pltpu.SemaphoreType.DMA,
            pltpu.SemaphoreType.DMA,
        ],
        name=name,
        compiler_params=pltpu.CompilerParams(needs_layout_passes=False),
    )
    def k(td_hbm, ts_hbm, eq_hbm, dst_hbm, src_hbm, zero_hbm, out_hbm,
          dvi, svi, rdv, rsv, eqv, msgv, aggsh, sid_, sis_, sd, ss, se):
        cid = lax.axis_index("c")
        sid = lax.axis_index("s")
        wid = cid * NS + sid
        base = wid * EW

        @pl.when(sid == 0)
        def _zero():
            pltpu.sync_copy(zero_hbm, aggsh)

        if conv1:
            def zrow(r, rc):
                for c in range(1, 8):
                    msgv[r, pl.ds(c * 16, 16)] = jnp.zeros((16,), jnp.float32)
                return rc
            lax.fori_loop(0, BLK, zrow, 0)

        plsc.subcore_barrier()

        def block(k_, carry):
            j1 = k_ - 1
            j2 = k_ - 2

            @pl.when(jnp.logical_and(j1 >= 0, j1 < NB))
            def _w_idx():
                s3 = lax.rem(j1, 3)
                off = pl.multiple_of(base + j1 * BLK, 8)
                pltpu.make_async_copy(dst_hbm.at[pl.ds(off, BLK)],
                                      dvi.at[s3], sid_).wait()
                pltpu.make_async_copy(src_hbm.at[pl.ds(off, BLK)],
                                      svi.at[s3], sis_).wait()

            @pl.when(k_ < NB)
            def _i_idx():
                s3 = lax.rem(k_, 3)
                off = pl.multiple_of(base + k_ * BLK, 8)
                pltpu.async_copy(dst_hbm.at[pl.ds(off, BLK)], dvi.at[s3],
                                 sid_)
                pltpu.async_copy(src_hbm.at[pl.ds(off, BLK)], svi.at[s3],
                                 sis_)

            @pl.when(j2 >= 0)
            def _w_gat():
                s3 = lax.rem(j2, 3)
                s2 = lax.rem(j2, 2)
                off = pl.multiple_of(base + j2 * BLK, 8)
                pltpu.make_async_copy(td_hbm.at[dvi.at[s3]], rdv.at[s2],
                                      sd).wait()
                pltpu.make_async_copy(ts_hbm.at[svi.at[s3]], rsv.at[s2],
                                      ss).wait()
                pltpu.make_async_copy(eq_hbm.at[pl.ds(off, BLK)], eqv.at[s2],
                                      se).wait()

            @pl.when(jnp.logical_and(j1 >= 0, j1 < NB))
            def _i_gat():
                s3 = lax.rem(j1, 3)
                s2 = lax.rem(j1, 2)
                off = pl.multiple_of(base + j1 * BLK, 8)
                pltpu.async_copy(td_hbm.at[dvi.at[s3]], rdv.at[s2], sd)
                pltpu.async_copy(ts_hbm.at[svi.at[s3]], rsv.at[s2], ss)
                pltpu.async_copy(eq_hbm.at[pl.ds(off, BLK)], eqv.at[s2], se)

            @pl.when(j2 >= 0)
            def _c():
                s3 = lax.rem(j2, 3)
                s2 = lax.rem(j2, 2)

                @plsc.parallel_loop(0, BLK)
                def row(r):
                    for c in range(CH):
                        lo = c * 16
                        if conv1:
                            f = (rdv[s2, r, pl.ds(0, 16)]
                                 + rsv[s2, r, pl.ds(0, 16)]
                                 + eqv[s2, r, pl.ds(0, 16)])
                            s = (rdv[s2, r, pl.ds(64, 16)]
                                 + rsv[s2, r, pl.ds(64, 16)]
                                 + eqv[s2, r, pl.ds(16, 16)])
                        else:
                            wd_ = rdv[s2, r, pl.ds(lo, 16)]
                            ws_ = rsv[s2, r, pl.ds(lo, 16)]
                            we_ = eqv[s2, r, pl.ds(lo, 16)]
                            hi = jnp.int32(-65536)

                            def bflo(w):
                                return plsc.bitcast(
                                    lax.shift_left(w, 16), jnp.float32)

                            def bfhi(w):
                                return plsc.bitcast(
                                    jnp.bitwise_and(w, hi), jnp.float32)

                            f = bflo(wd_) + bflo(ws_) + bflo(we_)
                            s = bfhi(wd_) + bfhi(ws_) + bfhi(we_)
                        msgv[r, pl.ds(lo, 16)] = _gate(f, s)
                pltpu.sync_copy(msgv, aggsh.at[dvi.at[s3]], add=True)

            return carry

        lax.fori_loop(0, NB + 2, block, 0)
        plsc.subcore_barrier()

        @pl.when(sid == 0)
        def _flush():
            pltpu.sync_copy(aggsh, out_hbm.at[cid])

    return k


def _pack16(f, s):
    bfi = lax.bitcast_convert_type(
        f.astype(jnp.bfloat16).astype(jnp.float32), jnp.int32)
    bsi = lax.bitcast_convert_type(
        s.astype(jnp.bfloat16).astype(jnp.float32), jnp.int32)
    return jnp.bitwise_or(jnp.bitwise_and(bsi, jnp.int32(-65536)),
                          lax.shift_right_logical(bfi, 16))


def _mm_body(x_ref, w_ref, b_ref, o_ref):
    acc = jnp.dot(x_ref[...], w_ref[...], preferred_element_type=jnp.float32)
    acc = acc + b_ref[...]
    if o_ref.dtype == jnp.int32:
        n2 = acc.shape[1] // 2
        o_ref[...] = _pack16(acc[:, :n2], acc[:, n2:])
    else:
        o_ref[...] = acc.astype(o_ref.dtype)


def _mm(x, w, b, blk, out_dt=jnp.float32):
    m, k = x.shape
    n = w.shape[1]
    no = n // 2 if out_dt == jnp.int32 else n
    return pl.pallas_call(
        _mm_body,
        grid=(m // blk,),
        in_specs=[
            pl.BlockSpec((blk, k), lambda i: (i, 0)),
            pl.BlockSpec((k, n), lambda i: (0, 0)),
            pl.BlockSpec((1, n), lambda i: (0, 0)),
        ],
        out_specs=pl.BlockSpec((blk, no), lambda i: (i, 0)),
        out_shape=jax.ShapeDtypeStruct((m, no), out_dt),
    )(x, w, b.reshape(1, n))


def _finish_body(ncut, x_ref, aa_ref, ab_ref, wp_ref, bp_ref, wd_ref, ws_ref,
                 h_ref, td_ref, ts_ref):
    h1 = x_ref[...] + aa_ref[:, :ncut] + ab_ref[:, :ncut]
    h = jnp.maximum(
        jnp.dot(h1, wp_ref[...], preferred_element_type=jnp.float32)
        + bp_ref[...], 0.0)
    h_ref[...] = h
    td = jnp.dot(h, wd_ref[...], preferred_element_type=jnp.float32)
    ts = jnp.dot(h, ws_ref[...], preferred_element_type=jnp.float32)
    td_ref[...] = _pack16(td[:, :HID], td[:, HID:])
    ts_ref[...] = _pack16(ts[:, :HID], ts[:, HID:])


def _finish(x, aggA, aggB, wp, bp, wd, ws, blk):
    m, d = x.shape
    wa = aggA.shape[1]
    kp, n = wp.shape
    n2 = wd.shape[1]
    return pl.pallas_call(
        functools.partial(_finish_body, d),
        grid=(m // blk,),
        in_specs=[
            pl.BlockSpec((blk, d), lambda i: (i, 0)),
            pl.BlockSpec((blk, wa), lambda i: (i, 0)),
            pl.BlockSpec((blk, wa), lambda i: (i, 0)),
            pl.BlockSpec((kp, n), lambda i: (0, 0)),
            pl.BlockSpec((1, n), lambda i: (0, 0)),
            pl.BlockSpec((n, n2), lambda i: (0, 0)),
            pl.BlockSpec((n, n2), lambda i: (0, 0)),
        ],
        out_specs=[
            pl.BlockSpec((blk, n), lambda i: (i, 0)),
            pl.BlockSpec((blk, n2 // 2), lambda i: (i, 0)),
            pl.BlockSpec((blk, n2 // 2), lambda i: (i, 0)),
        ],
        out_shape=[
            jax.ShapeDtypeStruct((m, n), jnp.float32),
            jax.ShapeDtypeStruct((m, n2 // 2), jnp.int32),
            jax.ShapeDtypeStruct((m, n2 // 2), jnp.int32),
        ],
    )(x, aggA, aggB, wp, bp.reshape(1, n), wd, ws)


def _final_body(h_ref, aa_ref, ab_ref, bt_ref, wfc_ref, bfc_ref, wh_ref,
                bh_ref, o_ref, sacc, cacc):
    i = pl.program_id(0)
    nblk = pl.num_programs(0)

    @pl.when(i == 0)
    def _init():
        sacc[...] = jnp.zeros_like(sacc)
        cacc[...] = jnp.zeros_like(cacc)

    h3 = jnp.maximum(h_ref[...] + aa_ref[...] + ab_ref[...], 0.0)
    ids = lax.broadcasted_iota(jnp.int32, (h3.shape[0], NUM_GRAPHS), 1)
    oh = (bt_ref[...] == ids).astype(jnp.float32)
    sacc[...] += lax.dot_general(oh, h3, (((0,), (0,)), ((), ())),
                                 preferred_element_type=jnp.float32)
    cnt = jnp.sum(oh, axis=0)
    cacc[...] += jnp.broadcast_to(cnt[:, None], cacc.shape)

    @pl.when(i == nblk - 1)
    def _emit():
        pooled = sacc[...] / jnp.maximum(cacc[...], 1.0)
        g = jnp.maximum(
            jnp.dot(pooled, wfc_ref[...], preferred_element_type=jnp.float32)
            + bfc_ref[...], 0.0)
        o_ref[...] = (jnp.dot(g, wh_ref[...], preferred_element_type=jnp.float32)
                      + bh_ref[...])


def _final(h, aggA, aggB, batch2d, wfc, bfc, wh, bh, blk):
    m = h.shape[0]
    nh = wh.shape[1]
    return pl.pallas_call(
        _final_body,
        grid=(m // blk,),
        in_specs=[
            pl.BlockSpec((blk, HID), lambda i: (i, 0)),
            pl.BlockSpec((blk, HID), lambda i: (i, 0)),
            pl.BlockSpec((blk, HID), lambda i: (i, 0)),
            pl.BlockSpec((blk, 1), lambda i: (i, 0)),
            pl.BlockSpec((HID, HID), lambda i: (0, 0)),
            pl.BlockSpec((1, HID), lambda i: (0, 0)),
            pl.BlockSpec((HID, nh), lambda i: (0, 0)),
            pl.BlockSpec((1, nh), lambda i: (0, 0)),
        ],
        out_specs=pl.BlockSpec((NUM_GRAPHS, nh), lambda i: (0, 0)),
        out_shape=jax.ShapeDtypeStruct((NUM_GRAPHS, nh), jnp.float32),
        scratch_shapes=[
            pltpu.VMEM((NUM_GRAPHS, HID), jnp.float32),
            pltpu.VMEM((NUM_GRAPHS, HID), jnp.float32),
        ],
    )(h, aggA, aggB, batch2d, wfc, bfc.reshape(1, HID), wh, bh.reshape(1, nh))


def kernel(x, edge_index, edge_attr, batch,
           Wf1, bf1, Ws1, bs1, Wp, bp,
           Wf2, bf2, Ws2, bs2, Wf3, bf3, Ws3, bs3,
           Wfc, bfc, Wh, bh):
    f32 = jnp.float32
    dst = edge_index[1]
    src = edge_index[0]

    w1d = jnp.zeros((NODE_DIM, HID), f32)
    w1d = w1d.at[:, 0:3].set(Wf1[0:3]).at[:, 64:67].set(Ws1[0:3])
    w1s = jnp.zeros((NODE_DIM, HID), f32)
    w1s = w1s.at[:, 0:3].set(Wf1[3:6]).at[:, 64:67].set(Ws1[3:6])
    w1e = jnp.zeros((EDGE_DIM, 32), f32)
    w1e = w1e.at[:, 0:3].set(Wf1[6:38]).at[:, 16:19].set(Ws1[6:38])
    b1q = jnp.zeros((32,), f32).at[0:3].set(bf1).at[16:19].set(bs1)

    wd2 = jnp.concatenate([Wf2[0:HID], Ws2[0:HID]], axis=1)
    ws2 = jnp.concatenate([Wf2[HID:2 * HID], Ws2[HID:2 * HID]], axis=1)
    we2 = jnp.concatenate([Wf2[2 * HID:], Ws2[2 * HID:]], axis=1)
    bq2 = jnp.concatenate([bf2, bs2])
    wd3 = jnp.concatenate([Wf3[0:HID], Ws3[0:HID]], axis=1)
    ws3 = jnp.concatenate([Wf3[HID:2 * HID], Ws3[HID:2 * HID]], axis=1)
    we3 = jnp.concatenate([Wf3[2 * HID:], Ws3[2 * HID:]], axis=1)
    bq3 = jnp.concatenate([bf3, bs3])

    z128 = jnp.zeros((N, HID), f32)

    t1d = _mm(x, w1d, jnp.zeros((HID,), f32), 1000)
    t1s = _mm(x, w1s, jnp.zeros((HID,), f32), 1000)
    eq1 = _mm(edge_attr, w1e, b1q, 4000)
    agg1 = _make_edge_sc(True, "edge_sc_conv1")(t1d, t1s, eq1, dst, src, z128)

    h, td2, ts2 = _finish(x, agg1[0], agg1[1], Wp, bp, wd2, ws2, 2000)
    eq2 = _mm(edge_attr, we2, bq2, 4000, jnp.int32)
    agg2 = _make_edge_sc(False, "edge_sc_hid")(td2, ts2, eq2, dst, src, z128)

    h2, td3, ts3 = _finish_hid(h, agg2[0], agg2[1], wd3, ws3, 2000)
    eq3 = _mm(edge_attr, we3, bq3, 4000, jnp.int32)
    agg3 = _make_edge_sc(False, "edge_sc_hid")(td3, ts3, eq3, dst, src, z128)

    return _final(h2, agg3[0], agg3[1], batch.reshape(N, 1),
                  Wfc, bfc, Wh, bh, 1000)


def _finish_hid_body(h_ref, aa_ref, ab_ref, wd_ref, ws_ref,
                     h2_ref, td_ref, ts_ref):
    h2 = jnp.maximum(h_ref[...] + aa_ref[...] + ab_ref[...], 0.0)
    h2_ref[...] = h2
    td = jnp.dot(h2, wd_ref[...], preferred_element_type=jnp.float32)
    ts = jnp.dot(h2, ws_ref[...], preferred_element_type=jnp.float32)
    td_ref[...] = _pack16(td[:, :HID], td[:, HID:])
    ts_ref[...] = _pack16(ts[:, :HID], ts[:, HID:])


def _finish_hid(h, aggA, aggB, wd, ws, blk):
    m = h.shape[0]
    n2 = wd.shape[1]
    return pl.pallas_call(
        _finish_hid_body,
        grid=(m // blk,),
        in_specs=[
            pl.BlockSpec((blk, HID), lambda i: (i, 0)),
            pl.BlockSpec((blk, HID), lambda i: (i, 0)),
            pl.BlockSpec((blk, HID), lambda i: (i, 0)),
            pl.BlockSpec((HID, n2), lambda i: (0, 0)),
            pl.BlockSpec((HID, n2), lambda i: (0, 0)),
        ],
        out_specs=[
            pl.BlockSpec((blk, HID), lambda i: (i, 0)),
            pl.BlockSpec((blk, n2 // 2), lambda i: (i, 0)),
            pl.BlockSpec((blk, n2 // 2), lambda i: (i, 0)),
        ],
        out_shape=[
            jax.ShapeDtypeStruct((m, HID), jnp.float32),
            jax.ShapeDtypeStruct((m, n2 // 2), jnp.int32),
            jax.ShapeDtypeStruct((m, n2 // 2), jnp.int32),
        ],
    )(h, aggA, aggB, wd, ws)


# speedup vs baseline: 4.5229x; 1.0111x over previous
import functools

import jax
import jax.numpy as jnp
from jax import lax
from jax.experimental import pallas as pl
from jax.experimental.pallas import tpu as pltpu
from jax.experimental.pallas import tpu_sc as plsc

N = 10000
E = 320000
NODE_DIM = 3
EDGE_DIM = 32
HID = 128
NUM_GRAPHS = 64

NC = 2
NS = 16
NW = NC * NS
EW = E // NW
BLK = 40
NB = EW // BLK

_LOG1P_C = (
    6.944574454165033e-05, 0.9962619482337974, -0.46644243862758844,
    0.2186654836622489, -0.05545931374208889,
)


def _log1p01(a):
    acc = jnp.full_like(a, _LOG1P_C[-1])
    for c in _LOG1P_C[-2::-1]:
        acc = acc * a + jnp.float32(c)
    return acc


def _gate(f, s):
    t = 1.0 + jnp.exp(-f)
    sp = jnp.maximum(s, 0.0) + _log1p01(jnp.exp(-jnp.abs(s)))
    return sp / t


@functools.lru_cache(maxsize=None)
def _make_edge_sc(conv1, name):
    CH = 1 if conv1 else 8
    if conv1:
        gshape = [(2, BLK, 128), (2, BLK, 128), (2, BLK, 32)]
        gdt = jnp.float32
    else:
        gshape = [(2, BLK, 128), (2, BLK, 128), (2, BLK, 128)]
        gdt = jnp.int32
    mesh = plsc.VectorSubcoreMesh(core_axis_name="c", subcore_axis_name="s",
                                  num_cores=NC, num_subcores=NS)

    @functools.partial(
        pl.kernel,
        out_type=jax.ShapeDtypeStruct((NC, N, HID), jnp.float32),
        mesh=mesh,
        scratch_types=[
            pltpu.VMEM((3, BLK), jnp.int32),
            pltpu.VMEM((3, BLK), jnp.int32),
            pltpu.VMEM(gshape[0], gdt),
            pltpu.VMEM(gshape[1], gdt),
            pltpu.VMEM(gshape[2], gdt),
            pltpu.VMEM((BLK, HID), jnp.float32),
            pltpu.VMEM_SHARED((N, HID), jnp.float32),
            pltpu.SemaphoreType.DMA,
            pltpu.SemaphoreType.DMA,
            pltpu.SemaphoreType.DMA,
            pltpu.SemaphoreType.DMA,
            pltpu.SemaphoreType.DMA,
        ],
        name=name,
        compiler_params=pltpu.CompilerParams(needs_layout_passes=False),
    )
    def k(td_hbm, ts_hbm, eq_hbm, dst_hbm, src_hbm, zero_hbm, out_hbm,
          dvi, svi, rdv, rsv, eqv, msgv, aggsh, sid_, sis_, sd, ss, se):
        cid = lax.axis_index("c")
        sid = lax.axis_index("s")
        wid = cid * NS + sid
        base = wid * EW

        @pl.when(sid == 0)
        def _zero():
            pltpu.sync_copy(zero_hbm, aggsh)

        if conv1:
            def zrow(r, rc):
                for c in range(1, 8):
                    msgv[r, pl.ds(c * 16, 16)] = jnp.zeros((16,), jnp.float32)
                return rc
            lax.fori_loop(0, BLK, zrow, 0)

        plsc.subcore_barrier()

        def block(k_, carry):
            j1 = k_ - 1
            j2 = k_ - 2

            @pl.when(jnp.logical_and(j1 >= 0, j1 < NB))
            def _w_idx():
                s3 = lax.rem(j1, 3)
                off = pl.multiple_of(base + j1 * BLK, 8)
                pltpu.make_async_copy(dst_hbm.at[pl.ds(off, BLK)],
                                      dvi.at[s3], sid_).wait()
                pltpu.make_async_copy(src_hbm.at[pl.ds(off, BLK)],
                                      svi.at[s3], sis_).wait()

            @pl.when(k_ < NB)
            def _i_idx():
                s3 = lax.rem(k_, 3)
                off = pl.multiple_of(base + k_ * BLK, 8)
                pltpu.async_copy(dst_hbm.at[pl.ds(off, BLK)], dvi.at[s3],
                                 sid_)
                pltpu.async_copy(src_hbm.at[pl.ds(off, BLK)], svi.at[s3],
                                 sis_)

            @pl.when(j2 >= 0)
            def _w_gat():
                s3 = lax.rem(j2, 3)
                s2 = lax.rem(j2, 2)
                off = pl.multiple_of(base + j2 * BLK, 8)
                pltpu.make_async_copy(td_hbm.at[dvi.at[s3]], rdv.at[s2],
                                      sd).wait()
                pltpu.make_async_copy(ts_hbm.at[svi.at[s3]], rsv.at[s2],
                                      ss).wait()
                pltpu.make_async_copy(eq_hbm.at[pl.ds(off, BLK)], eqv.at[s2],
                                      se).wait()

            @pl.when(jnp.logical_and(j1 >= 0, j1 < NB))
            def _i_gat():
                s3 = lax.rem(j1, 3)
                s2 = lax.rem(j1, 2)
                off = pl.multiple_of(base + j1 * BLK, 8)
                pltpu.async_copy(td_hbm.at[dvi.at[s3]], rdv.at[s2], sd)
                pltpu.async_copy(ts_hbm.at[svi.at[s3]], rsv.at[s2], ss)
                pltpu.async_copy(eq_hbm.at[pl.ds(off, BLK)], eqv.at[s2], se)

            @pl.when(j2 >= 0)
            def _c():
                s3 = lax.rem(j2, 3)
                s2 = lax.rem(j2, 2)

                @plsc.parallel_loop(0, BLK, unroll=2)
                def row(r):
                    for c in range(CH):
                        lo = c * 16
                        if conv1:
                            f = (rdv[s2, r, pl.ds(0, 16)]
                                 + rsv[s2, r, pl.ds(0, 16)]
                                 + eqv[s2, r, pl.ds(0, 16)])
                            s = (rdv[s2, r, pl.ds(64, 16)]
                                 + rsv[s2, r, pl.ds(64, 16)]
                                 + eqv[s2, r, pl.ds(16, 16)])
                        else:
                            wd_ = rdv[s2, r, pl.ds(lo, 16)]
                            ws_ = rsv[s2, r, pl.ds(lo, 16)]
                            we_ = eqv[s2, r, pl.ds(lo, 16)]
                            hi = jnp.int32(-65536)

                            def bflo(w):
                                return plsc.bitcast(
                                    lax.shift_left(w, 16), jnp.float32)

                            def bfhi(w):
                                return plsc.bitcast(
                                    jnp.bitwise_and(w, hi), jnp.float32)

                            f = bflo(wd_) + bflo(ws_) + bflo(we_)
                            s = bfhi(wd_) + bfhi(ws_) + bfhi(we_)
                        msgv[r, pl.ds(lo, 16)] = _gate(f, s)
                pltpu.sync_copy(msgv, aggsh.at[dvi.at[s3]], add=True)

            return carry

        lax.fori_loop(0, NB + 2, block, 0)
        plsc.subcore_barrier()

        @pl.when(sid == 0)
        def _flush():
            pltpu.sync_copy(aggsh, out_hbm.at[cid])

    return k


def _pack16(f, s):
    bfi = lax.bitcast_convert_type(
        f.astype(jnp.bfloat16).astype(jnp.float32), jnp.int32)
    bsi = lax.bitcast_convert_type(
        s.astype(jnp.bfloat16).astype(jnp.float32), jnp.int32)
    return jnp.bitwise_or(jnp.bitwise_and(bsi, jnp.int32(-65536)),
                          lax.shift_right_logical(bfi, 16))


def _mm_body(x_ref, w_ref, b_ref, o_ref):
    acc = jnp.dot(x_ref[...], w_ref[...], preferred_element_type=jnp.float32)
    acc = acc + b_ref[...]
    if o_ref.dtype == jnp.int32:
        n2 = acc.shape[1] // 2
        o_ref[...] = _pack16(acc[:, :n2], acc[:, n2:])
    else:
        o_ref[...] = acc.astype(o_ref.dtype)


def _mm(x, w, b, blk, out_dt=jnp.float32):
    m, k = x.shape
    n = w.shape[1]
    no = n // 2 if out_dt == jnp.int32 else n
    return pl.pallas_call(
        _mm_body,
        grid=(m // blk,),
        in_specs=[
            pl.BlockSpec((blk, k), lambda i: (i, 0)),
            pl.BlockSpec((k, n), lambda i: (0, 0)),
            pl.BlockSpec((1, n), lambda i: (0, 0)),
        ],
        out_specs=pl.BlockSpec((blk, no), lambda i: (i, 0)),
        out_shape=jax.ShapeDtypeStruct((m, no), out_dt),
    )(x, w, b.reshape(1, n))


def _finish_body(ncut, x_ref, aa_ref, ab_ref, wp_ref, bp_ref, wd_ref, ws_ref,
                 h_ref, td_ref, ts_ref):
    h1 = x_ref[...] + aa_ref[:, :ncut] + ab_ref[:, :ncut]
    h = jnp.maximum(
        jnp.dot(h1, wp_ref[...], preferred_element_type=jnp.float32)
        + bp_ref[...], 0.0)
    h_ref[...] = h
    td = jnp.dot(h, wd_ref[...], preferred_element_type=jnp.float32)
    ts = jnp.dot(h, ws_ref[...], preferred_element_type=jnp.float32)
    td_ref[...] = _pack16(td[:, :HID], td[:, HID:])
    ts_ref[...] = _pack16(ts[:, :HID], ts[:, HID:])


def _finish(x, aggA, aggB, wp, bp, wd, ws, blk):
    m, d = x.shape
    wa = aggA.shape[1]
    kp, n = wp.shape
    n2 = wd.shape[1]
    return pl.pallas_call(
        functools.partial(_finish_body, d),
        grid=(m // blk,),
        in_specs=[
            pl.BlockSpec((blk, d), lambda i: (i, 0)),
            pl.BlockSpec((blk, wa), lambda i: (i, 0)),
            pl.BlockSpec((blk, wa), lambda i: (i, 0)),
            pl.BlockSpec((kp, n), lambda i: (0, 0)),
            pl.BlockSpec((1, n), lambda i: (0, 0)),
            pl.BlockSpec((n, n2), lambda i: (0, 0)),
            pl.BlockSpec((n, n2), lambda i: (0, 0)),
        ],
        out_specs=[
            pl.BlockSpec((blk, n), lambda i: (i, 0)),
            pl.BlockSpec((blk, n2 // 2), lambda i: (i, 0)),
            pl.BlockSpec((blk, n2 // 2), lambda i: (i, 0)),
        ],
        out_shape=[
            jax.ShapeDtypeStruct((m, n), jnp.float32),
            jax.ShapeDtypeStruct((m, n2 // 2), jnp.int32),
            jax.ShapeDtypeStruct((m, n2 // 2), jnp.int32),
        ],
    )(x, aggA, aggB, wp, bp.reshape(1, n), wd, ws)


def _final_body(h_ref, aa_ref, ab_ref, bt_ref, wfc_ref, bfc_ref, wh_ref,
                bh_ref, o_ref, sacc, cacc):
    i = pl.program_id(0)
    nblk = pl.num_programs(0)

    @pl.when(i == 0)
    def _init():
        sacc[...] = jnp.zeros_like(sacc)
        cacc[...] = jnp.zeros_like(cacc)

    h3 = jnp.maximum(h_ref[...] + aa_ref[...] + ab_ref[...], 0.0)
    ids = lax.broadcasted_iota(jnp.int32, (h3.shape[0], NUM_GRAPHS), 1)
    oh = (bt_ref[...] == ids).astype(jnp.float32)
    sacc[...] += lax.dot_general(oh, h3, (((0,), (0,)), ((), ())),
                                 preferred_element_type=jnp.float32)
    cnt = jnp.sum(oh, axis=0)
    cacc[...] += jnp.broadcast_to(cnt[:, None], cacc.shape)

    @pl.when(i == nblk - 1)
    def _emit():
        pooled = sacc[...] / jnp.maximum(cacc[...], 1.0)
        g = jnp.maximum(
            jnp.dot(pooled, wfc_ref[...], preferred_element_type=jnp.float32)
            + bfc_ref[...], 0.0)
        o_ref[...] = (jnp.dot(g, wh_ref[...], preferred_element_type=jnp.float32)
                      + bh_ref[...])


def _final(h, aggA, aggB, batch2d, wfc, bfc, wh, bh, blk):
    m = h.shape[0]
    nh = wh.shape[1]
    return pl.pallas_call(
        _final_body,
        grid=(m // blk,),
        in_specs=[
            pl.BlockSpec((blk, HID), lambda i: (i, 0)),
            pl.BlockSpec((blk, HID), lambda i: (i, 0)),
            pl.BlockSpec((blk, HID), lambda i: (i, 0)),
            pl.BlockSpec((blk, 1), lambda i: (i, 0)),
            pl.BlockSpec((HID, HID), lambda i: (0, 0)),
            pl.BlockSpec((1, HID), lambda i: (0, 0)),
            pl.BlockSpec((HID, nh), lambda i: (0, 0)),
            pl.BlockSpec((1, nh), lambda i: (0, 0)),
        ],
        out_specs=pl.BlockSpec((NUM_GRAPHS, nh), lambda i: (0, 0)),
        out_shape=jax.ShapeDtypeStruct((NUM_GRAPHS, nh), jnp.float32),
        scratch_shapes=[
            pltpu.VMEM((NUM_GRAPHS, HID), jnp.float32),
            pltpu.VMEM((NUM_GRAPHS, HID), jnp.float32),
        ],
    )(h, aggA, aggB, batch2d, wfc, bfc.reshape(1, HID), wh, bh.reshape(1, nh))


def kernel(x, edge_index, edge_attr, batch,
           Wf1, bf1, Ws1, bs1, Wp, bp,
           Wf2, bf2, Ws2, bs2, Wf3, bf3, Ws3, bs3,
           Wfc, bfc, Wh, bh):
    f32 = jnp.float32
    dst = edge_index[1]
    src = edge_index[0]

    w1d = jnp.zeros((NODE_DIM, HID), f32)
    w1d = w1d.at[:, 0:3].set(Wf1[0:3]).at[:, 64:67].set(Ws1[0:3])
    w1s = jnp.zeros((NODE_DIM, HID), f32)
    w1s = w1s.at[:, 0:3].set(Wf1[3:6]).at[:, 64:67].set(Ws1[3:6])
    w1e = jnp.zeros((EDGE_DIM, 32), f32)
    w1e = w1e.at[:, 0:3].set(Wf1[6:38]).at[:, 16:19].set(Ws1[6:38])
    b1q = jnp.zeros((32,), f32).at[0:3].set(bf1).at[16:19].set(bs1)

    wd2 = jnp.concatenate([Wf2[0:HID], Ws2[0:HID]], axis=1)
    ws2 = jnp.concatenate([Wf2[HID:2 * HID], Ws2[HID:2 * HID]], axis=1)
    we2 = jnp.concatenate([Wf2[2 * HID:], Ws2[2 * HID:]], axis=1)
    bq2 = jnp.concatenate([bf2, bs2])
    wd3 = jnp.concatenate([Wf3[0:HID], Ws3[0:HID]], axis=1)
    ws3 = jnp.concatenate([Wf3[HID:2 * HID], Ws3[HID:2 * HID]], axis=1)
    we3 = jnp.concatenate([Wf3[2 * HID:], Ws3[2 * HID:]], axis=1)
    bq3 = jnp.concatenate([bf3, bs3])

    z128 = jnp.zeros((N, HID), f32)

    t1d = _mm(x, w1d, jnp.zeros((HID,), f32), 1000)
    t1s = _mm(x, w1s, jnp.zeros((HID,), f32), 1000)
    eq1 = _mm(edge_attr, w1e, b1q, 4000)
    agg1 = _make_edge_sc(True, "edge_sc_conv1")(t1d, t1s, eq1, dst, src, z128)

    h, td2, ts2 = _finish(x, agg1[0], agg1[1], Wp, bp, wd2, ws2, 2000)
    eq2 = _mm(edge_attr, we2, bq2, 4000, jnp.int32)
    agg2 = _make_edge_sc(False, "edge_sc_hid")(td2, ts2, eq2, dst, src, z128)

    h2, td3, ts3 = _finish_hid(h, agg2[0], agg2[1], wd3, ws3, 2000)
    eq3 = _mm(edge_attr, we3, bq3, 4000, jnp.int32)
    agg3 = _make_edge_sc(False, "edge_sc_hid")(td3, ts3, eq3, dst, src, z128)

    return _final(h2, agg3[0], agg3[1], batch.reshape(N, 1),
                  Wfc, bfc, Wh, bh, 1000)


def _finish_hid_body(h_ref, aa_ref, ab_ref, wd_ref, ws_ref,
                     h2_ref, td_ref, ts_ref):
    h2 = jnp.maximum(h_ref[...] + aa_ref[...] + ab_ref[...], 0.0)
    h2_ref[...] = h2
    td = jnp.dot(h2, wd_ref[...], preferred_element_type=jnp.float32)
    ts = jnp.dot(h2, ws_ref[...], preferred_element_type=jnp.float32)
    td_ref[...] = _pack16(td[:, :HID], td[:, HID:])
    ts_ref[...] = _pack16(ts[:, :HID], ts[:, HID:])


def _finish_hid(h, aggA, aggB, wd, ws, blk):
    m = h.shape[0]
    n2 = wd.shape[1]
    return pl.pallas_call(
        _finish_hid_body,
        grid=(m // blk,),
        in_specs=[
            pl.BlockSpec((blk, HID), lambda i: (i, 0)),
            pl.BlockSpec((blk, HID), lambda i: (i, 0)),
            pl.BlockSpec((blk, HID), lambda i: (i, 0)),
            pl.BlockSpec((HID, n2), lambda i: (0, 0)),
            pl.BlockSpec((HID, n2), lambda i: (0, 0)),
        ],
        out_specs=[
            pl.BlockSpec((blk, HID), lambda i: (i, 0)),
            pl.BlockSpec((blk, n2 // 2), lambda i: (i, 0)),
            pl.BlockSpec((blk, n2 // 2), lambda i: (i, 0)),
        ],
        out_shape=[
            jax.ShapeDtypeStruct((m, HID), jnp.float32),
            jax.ShapeDtypeStruct((m, n2 // 2), jnp.int32),
            jax.ShapeDtypeStruct((m, n2 // 2), jnp.int32),
        ],
    )(h, aggA, aggB, wd, ws)
